# baseline (speedup 1.0000x reference)
import functools

import numpy as np
import jax
import jax.numpy as jnp
from jax import lax
from jax.experimental import pallas as pl
from jax.experimental.pallas import tpu as pltpu
from jax.experimental.pallas import tpu_sc as plsc

B = 4096
D = 64
W = 2 * D
QL = 20
DL = 200
VR = 500000
NC = 2
NS = 16
NW = NC * NS
RPW = B // NW
QIW = RPW * QL
DIW = RPW * DL
CH = 128
QCH = QIW // CH
DCH = DIW // CH
AR = 2 * RPW


def _seg_base(n_total: int, per_worker: int, seg_len: int) -> np.ndarray:
    pos = np.arange(n_total, dtype=np.int32)
    w = pos // per_worker
    row = (pos % per_worker) // seg_len
    return (w // NC) * AR + 2 * row


_QBASE = _seg_base(B * QL, QIW, QL)
_DBASE = _seg_base(B * DL, DIW, DL)


def _sc_pool(qt2, dt2, gq, gd, qpat, dpat, zeros):
    mesh = plsc.VectorSubcoreMesh(core_axis_name="c", subcore_axis_name="s")

    @functools.partial(
        pl.kernel,
        out_type=[
            jax.ShapeDtypeStruct((NW * AR, W), jnp.float32),
            jax.ShapeDtypeStruct((NW * AR, W), jnp.float32),
        ],
        mesh=mesh,
        compiler_params=pltpu.CompilerParams(use_tc_tiling_on_sc=False),
        scratch_types=[
            pltpu.VMEM((QIW,), jnp.int32),
            pltpu.VMEM((DIW,), jnp.int32),
            pltpu.VMEM((QCH, CH), jnp.int32),
            pltpu.VMEM((DCH, CH), jnp.int32),
            pltpu.VMEM((CH, W), jnp.float32),
            pltpu.VMEM_SHARED((NS * AR, W), jnp.float32),
        ],
    )
    def sc_kernel(qt_hbm, dt_hbm, qi_hbm, di_hbm, qp_hbm, dp_hbm, z_hbm,
                  qsum_hbm, dsum_hbm,
                  qidx_v, didx_v, qpat_v, dpat_v, buf_v, sh):
        sid = lax.axis_index("s")
        wid = sid * NC + lax.axis_index("c")

        pltpu.sync_copy(qi_hbm.at[pl.ds(wid * QIW, QIW)], qidx_v)
        pltpu.sync_copy(di_hbm.at[pl.ds(wid * DIW, DIW)], didx_v)
        pltpu.sync_copy(qp_hbm.at[wid], qpat_v)
        pltpu.sync_copy(dp_hbm.at[wid], dpat_v)
        pltpu.sync_copy(z_hbm, sh.at[pl.ds(sid * AR, AR)])

        @pl.loop(0, QCH)
        def _(c):
            pltpu.sync_copy(qt_hbm.at[qidx_v.at[pl.ds(c * CH, CH)]], buf_v)
            pltpu.sync_copy(buf_v, sh.at[qpat_v.at[c]], add=True)

        pltpu.sync_copy(sh.at[pl.ds(sid * AR, AR)], qsum_hbm.at[pl.ds(wid * AR, AR)])
        pltpu.sync_copy(z_hbm, sh.at[pl.ds(sid * AR, AR)])

        @pl.loop(0, DCH)
        def _(c):
            pltpu.sync_copy(dt_hbm.at[didx_v.at[pl.ds(c * CH, CH)]], buf_v)
            pltpu.sync_copy(buf_v, sh.at[dpat_v.at[c]], add=True)

        pltpu.sync_copy(sh.at[pl.ds(sid * AR, AR)], dsum_hbm.at[pl.ds(wid * AR, AR)])

    return sc_kernel(qt2, dt2, gq, gd, qpat, dpat, zeros)


def _tc_finish(q_sum, d_sum):
    def body(qs_ref, ds_ref, o_ref):
        qx = qs_ref[...]
        dx = ds_ref[...]
        q = jnp.tanh((qx[:, 0:D] + qx[:, W + D:2 * W]) * (1.0 / QL))
        d = jnp.tanh((dx[:, 0:D] + dx[:, W + D:2 * W]) * (1.0 / DL))
        qn = jnp.maximum(jnp.sqrt(jnp.sum(q * q, axis=1, keepdims=True)), 1e-12)
        dn = jnp.maximum(jnp.sqrt(jnp.sum(d * d, axis=1, keepdims=True)), 1e-12)
        o_ref[...] = jnp.sum((q / qn) * (d / dn), axis=1)

    return pl.pallas_call(
        body,
        out_shape=jax.ShapeDtypeStruct((B,), jnp.float32),
    )(q_sum, d_sum)


def kernel(qs, ds, q_table, d_table):
    qt2 = q_table.reshape(VR, W)
    dt2 = d_table.reshape(VR, W)
    qsf = qs.reshape(-1)
    dsf = ds.reshape(-1)
    gq = qsf >> 1
    gd = dsf >> 1
    qpat = (jnp.asarray(_QBASE) + (qsf & 1)).reshape(NW, QCH, CH)
    dpat = (jnp.asarray(_DBASE) + (dsf & 1)).reshape(NW, DCH, CH)
    zeros = jnp.zeros((AR, W), jnp.float32)
    q_sum, d_sum = _sc_pool(qt2, dt2, gq, gd, qpat, dpat, zeros)
    return _tc_finish(q_sum.reshape(B, 2 * W), d_sum.reshape(B, 2 * W))


# speedup vs baseline: 1.1114x; 1.1114x over previous
import functools

import numpy as np
import jax
import jax.numpy as jnp
from jax import lax
from jax.experimental import pallas as pl
from jax.experimental.pallas import tpu as pltpu
from jax.experimental.pallas import tpu_sc as plsc

B = 4096
D = 64
QL = 20
DL = 200
NC = 2
NS = 16
NW = NC * NS
RPW = B // NW
QIW = RPW * QL
DIW = RPW * DL
CH = 128
QCH = QIW // CH
DCH = DIW // CH


def _seg_pattern(n_chunks: int, seg_len: int) -> np.ndarray:
    pos = np.arange(n_chunks * CH, dtype=np.int32)
    base = (pos // seg_len).reshape(1, n_chunks, CH)
    offs = (np.arange(NS, dtype=np.int32) * RPW).reshape(NS, 1, 1)
    return base + offs


_QPAT = _seg_pattern(QCH, QL)
_DPAT = _seg_pattern(DCH, DL)


def _sc_pool(q_table, d_table, qs_flat, ds_flat, qpat, dpat, zeros):
    mesh = plsc.VectorSubcoreMesh(core_axis_name="c", subcore_axis_name="s")

    @functools.partial(
        pl.kernel,
        out_type=[
            jax.ShapeDtypeStruct((B, D), jnp.float32),
            jax.ShapeDtypeStruct((B, D), jnp.float32),
        ],
        mesh=mesh,
        compiler_params=pltpu.CompilerParams(use_tc_tiling_on_sc=False),
        scratch_types=[
            pltpu.VMEM((QIW,), jnp.int32),
            pltpu.VMEM((DIW,), jnp.int32),
            pltpu.VMEM((QCH, CH), jnp.int32),
            pltpu.VMEM((DCH, CH), jnp.int32),
            pltpu.VMEM((CH, D), jnp.float32),
            pltpu.VMEM_SHARED((NS * RPW, D), jnp.float32),
            pltpu.VMEM_SHARED((NS * RPW, D), jnp.float32),
        ],
    )
    def sc_kernel(qt_hbm, dt_hbm, qi_hbm, di_hbm, qp_hbm, dp_hbm, z_hbm,
                  qsum_hbm, dsum_hbm,
                  qidx_v, didx_v, qpat_v, dpat_v, buf_v, qsh, dsh):
        sid = lax.axis_index("s")
        wid = sid * NC + lax.axis_index("c")

        pltpu.sync_copy(qi_hbm.at[pl.ds(wid * QIW, QIW)], qidx_v)
        pltpu.sync_copy(di_hbm.at[pl.ds(wid * DIW, DIW)], didx_v)
        pltpu.sync_copy(qp_hbm.at[sid], qpat_v)
        pltpu.sync_copy(dp_hbm.at[sid], dpat_v)
        pltpu.sync_copy(z_hbm, qsh.at[pl.ds(sid * RPW, RPW)])
        pltpu.sync_copy(z_hbm, dsh.at[pl.ds(sid * RPW, RPW)])

        @pl.loop(0, QCH)
        def _(c):
            pltpu.sync_copy(qt_hbm.at[qidx_v.at[pl.ds(c * CH, CH)]], buf_v)
            pltpu.sync_copy(buf_v, qsh.at[qpat_v.at[c]], add=True)

        @pl.loop(0, DCH)
        def _(c):
            pltpu.sync_copy(dt_hbm.at[didx_v.at[pl.ds(c * CH, CH)]], buf_v)
            pltpu.sync_copy(buf_v, dsh.at[dpat_v.at[c]], add=True)

        pltpu.sync_copy(qsh.at[pl.ds(sid * RPW, RPW)], qsum_hbm.at[pl.ds(wid * RPW, RPW)])
        pltpu.sync_copy(dsh.at[pl.ds(sid * RPW, RPW)], dsum_hbm.at[pl.ds(wid * RPW, RPW)])

    return sc_kernel(q_table, d_table, qs_flat, ds_flat, qpat, dpat, zeros)


def _tc_finish(q_sum, d_sum):
    def body(qs_ref, ds_ref, o_ref):
        q = jnp.tanh(qs_ref[...] * (1.0 / QL))
        d = jnp.tanh(ds_ref[...] * (1.0 / DL))
        qn = jnp.maximum(jnp.sqrt(jnp.sum(q * q, axis=1, keepdims=True)), 1e-12)
        dn = jnp.maximum(jnp.sqrt(jnp.sum(d * d, axis=1, keepdims=True)), 1e-12)
        o_ref[...] = jnp.sum((q / qn) * (d / dn), axis=1)

    return pl.pallas_call(
        body,
        out_shape=jax.ShapeDtypeStruct((B,), jnp.float32),
    )(q_sum, d_sum)


def kernel(qs, ds, q_table, d_table):
    qs_flat = qs.reshape(-1)
    ds_flat = ds.reshape(-1)
    qpat = jnp.asarray(_QPAT)
    dpat = jnp.asarray(_DPAT)
    zeros = jnp.zeros((RPW, D), jnp.float32)
    q_sum, d_sum = _sc_pool(q_table, d_table, qs_flat, ds_flat, qpat, dpat, zeros)
    return _tc_finish(q_sum, d_sum)


# speedup vs baseline: 1.2203x; 1.0980x over previous
import functools

import numpy as np
import jax
import jax.numpy as jnp
from jax import lax
from jax.experimental import pallas as pl
from jax.experimental.pallas import tpu as pltpu
from jax.experimental.pallas import tpu_sc as plsc

B = 4096
D = 64
QL = 20
DL = 200
NC = 2
NS = 16
NW = NC * NS
RPW = B // NW
QIW = RPW * QL
DIW = RPW * DL
CH = 128
QCH = QIW // CH
DCH = DIW // CH
NB = 4


def _seg_pattern(n_chunks: int, seg_len: int) -> np.ndarray:
    pos = np.arange(n_chunks * CH, dtype=np.int32)
    base = (pos // seg_len).reshape(1, n_chunks, CH)
    offs = (np.arange(NS, dtype=np.int32) * RPW).reshape(NS, 1, 1)
    return base + offs


_QPAT = _seg_pattern(QCH, QL)
_DPAT = _seg_pattern(DCH, DL)


def _sc_pool(q_table, d_table, qs_flat, ds_flat, qpat, dpat, zeros):
    mesh = plsc.VectorSubcoreMesh(core_axis_name="c", subcore_axis_name="s")

    @functools.partial(
        pl.kernel,
        out_type=[
            jax.ShapeDtypeStruct((B, D), jnp.float32),
            jax.ShapeDtypeStruct((B, D), jnp.float32),
        ],
        mesh=mesh,
        compiler_params=pltpu.CompilerParams(use_tc_tiling_on_sc=False),
        scratch_types=[
            pltpu.VMEM((QIW,), jnp.int32),
            pltpu.VMEM((DIW,), jnp.int32),
            pltpu.VMEM((QCH, CH), jnp.int32),
            pltpu.VMEM((DCH, CH), jnp.int32),
            pltpu.VMEM((NB, CH, D), jnp.float32),
            pltpu.VMEM_SHARED((NS * RPW, D), jnp.float32),
            pltpu.VMEM_SHARED((NS * RPW, D), jnp.float32),
        ] + [pltpu.SemaphoreType.DMA] * NB,
    )
    def sc_kernel(qt_hbm, dt_hbm, qi_hbm, di_hbm, qp_hbm, dp_hbm, z_hbm,
                  qsum_hbm, dsum_hbm,
                  qidx_v, didx_v, qpat_v, dpat_v, buf_v, qsh, dsh, *gsems):
        sid = lax.axis_index("s")
        wid = sid * NC + lax.axis_index("c")

        pltpu.sync_copy(qi_hbm.at[pl.ds(wid * QIW, QIW)], qidx_v)
        pltpu.sync_copy(di_hbm.at[pl.ds(wid * DIW, DIW)], didx_v)
        pltpu.sync_copy(qp_hbm.at[sid], qpat_v)
        pltpu.sync_copy(dp_hbm.at[sid], dpat_v)
        pltpu.sync_copy(z_hbm, qsh.at[pl.ds(sid * RPW, RPW)])
        pltpu.sync_copy(z_hbm, dsh.at[pl.ds(sid * RPW, RPW)])

        def pooled(tab, idx_v, pat_v, acc, n_chunks):
            def start(c, b):
                pltpu.async_copy(
                    tab.at[idx_v.at[pl.ds(c * CH, CH)]], buf_v.at[b], gsems[b])

            def finish(c, b):
                pltpu.make_async_copy(
                    tab.at[idx_v.at[pl.ds(0, CH)]], buf_v.at[b], gsems[b]
                ).wait()
                pltpu.sync_copy(buf_v.at[b], acc.at[pat_v.at[c]], add=True)

            for b in range(NB):
                start(b, b)

            @pl.loop(0, n_chunks // NB - 1)
            def _(g):
                for b in range(NB):
                    finish(g * NB + b, b)
                    start((g + 1) * NB + b, b)

            for b in range(NB):
                finish((n_chunks // NB - 1) * NB + b, b)

        pooled(qt_hbm, qidx_v, qpat_v, qsh, QCH)
        pooled(dt_hbm, didx_v, dpat_v, dsh, DCH)

        pltpu.sync_copy(qsh.at[pl.ds(sid * RPW, RPW)], qsum_hbm.at[pl.ds(wid * RPW, RPW)])
        pltpu.sync_copy(dsh.at[pl.ds(sid * RPW, RPW)], dsum_hbm.at[pl.ds(wid * RPW, RPW)])

    return sc_kernel(q_table, d_table, qs_flat, ds_flat, qpat, dpat, zeros)


def _tc_finish(q_sum, d_sum):
    def body(qs_ref, ds_ref, o_ref):
        q = jnp.tanh(qs_ref[...] * (1.0 / QL))
        d = jnp.tanh(ds_ref[...] * (1.0 / DL))
        qn = jnp.maximum(jnp.sqrt(jnp.sum(q * q, axis=1, keepdims=True)), 1e-12)
        dn = jnp.maximum(jnp.sqrt(jnp.sum(d * d, axis=1, keepdims=True)), 1e-12)
        o_ref[...] = jnp.sum((q / qn) * (d / dn), axis=1)

    return pl.pallas_call(
        body,
        out_shape=jax.ShapeDtypeStruct((B,), jnp.float32),
    )(q_sum, d_sum)


def kernel(qs, ds, q_table, d_table):
    qs_flat = qs.reshape(-1)
    ds_flat = ds.reshape(-1)
    qpat = jnp.asarray(_QPAT)
    dpat = jnp.asarray(_DPAT)
    zeros = jnp.zeros((RPW, D), jnp.float32)
    q_sum, d_sum = _sc_pool(q_table, d_table, qs_flat, ds_flat, qpat, dpat, zeros)
    return _tc_finish(q_sum, d_sum)


# speedup vs baseline: 1.2329x; 1.0103x over previous
import functools

import numpy as np
import jax
import jax.numpy as jnp
from jax import lax
from jax.experimental import pallas as pl
from jax.experimental.pallas import tpu as pltpu
from jax.experimental.pallas import tpu_sc as plsc

B = 4096
D = 64
QL = 20
DL = 200
NC = 2
NS = 16
NW = NC * NS
RPW = B // NW
QIW = RPW * QL
DIW = RPW * DL
CH = 128
QCH = QIW // CH
DCH = DIW // CH
NB = 4


def _seg_pattern(n_chunks: int, seg_len: int) -> np.ndarray:
    pos = np.arange(n_chunks * CH, dtype=np.int32)
    base = (pos // seg_len).reshape(1, n_chunks, CH)
    offs = (np.arange(NS, dtype=np.int32) * RPW).reshape(NS, 1, 1)
    return base + offs


_QPAT = _seg_pattern(QCH, QL)
_DPAT = _seg_pattern(DCH, DL)


def _sc_pool(q_table, d_table, qs_flat, ds_flat, qpat, dpat, zeros):
    mesh = plsc.VectorSubcoreMesh(core_axis_name="c", subcore_axis_name="s")

    @functools.partial(
        pl.kernel,
        out_type=[
            jax.ShapeDtypeStruct((B, D), jnp.float32),
            jax.ShapeDtypeStruct((B, D), jnp.float32),
        ],
        mesh=mesh,
        compiler_params=pltpu.CompilerParams(use_tc_tiling_on_sc=False),
        scratch_types=[
            pltpu.VMEM((QIW,), jnp.int32),
            pltpu.VMEM((DIW,), jnp.int32),
            pltpu.VMEM((QCH, CH), jnp.int32),
            pltpu.VMEM((DCH, CH), jnp.int32),
            pltpu.VMEM((NB, CH, D), jnp.float32),
            pltpu.VMEM_SHARED((NS * RPW, D), jnp.float32),
            pltpu.VMEM_SHARED((NS * RPW, D), jnp.float32),
        ] + [pltpu.SemaphoreType.DMA] * (2 * NB),
    )
    def sc_kernel(qt_hbm, dt_hbm, qi_hbm, di_hbm, qp_hbm, dp_hbm, z_hbm,
                  qsum_hbm, dsum_hbm,
                  qidx_v, didx_v, qpat_v, dpat_v, buf_v, qsh, dsh, *sems):
        gsems, ssems = sems[:NB], sems[NB:]
        sid = lax.axis_index("s")
        wid = sid * NC + lax.axis_index("c")

        pltpu.sync_copy(qi_hbm.at[pl.ds(wid * QIW, QIW)], qidx_v)
        pltpu.sync_copy(di_hbm.at[pl.ds(wid * DIW, DIW)], didx_v)
        pltpu.sync_copy(qp_hbm.at[sid], qpat_v)
        pltpu.sync_copy(dp_hbm.at[sid], dpat_v)
        pltpu.sync_copy(z_hbm, qsh.at[pl.ds(sid * RPW, RPW)])
        pltpu.sync_copy(z_hbm, dsh.at[pl.ds(sid * RPW, RPW)])

        def pooled(tab, idx_v, pat_v, acc, n_chunks):
            def g_start(c, b):
                pltpu.async_copy(
                    tab.at[idx_v.at[pl.ds(c * CH, CH)]], buf_v.at[b], gsems[b])

            def g_wait(b):
                pltpu.make_async_copy(
                    tab.at[idx_v.at[pl.ds(0, CH)]], buf_v.at[b], gsems[b]
                ).wait()

            def s_start(c, b):
                pltpu.async_copy(
                    buf_v.at[b], acc.at[pat_v.at[c]], ssems[b], add=True)

            def s_wait(b):
                pltpu.make_async_copy(
                    buf_v.at[b], acc.at[pat_v.at[0]], ssems[b]).wait()

            for b in range(NB):
                g_start(b, b)

            @pl.loop(0, n_chunks // NB - 1)
            def _(g):
                for b in range(NB):
                    g_wait(b)
                    s_start(g * NB + b, b)
                for b in range(NB):
                    s_wait(b)
                    g_start((g + 1) * NB + b, b)

            last = (n_chunks // NB - 1) * NB
            for b in range(NB):
                g_wait(b)
                s_start(last + b, b)
            for b in range(NB):
                s_wait(b)

        pooled(qt_hbm, qidx_v, qpat_v, qsh, QCH)
        pooled(dt_hbm, didx_v, dpat_v, dsh, DCH)

        pltpu.sync_copy(qsh.at[pl.ds(sid * RPW, RPW)], qsum_hbm.at[pl.ds(wid * RPW, RPW)])
        pltpu.sync_copy(dsh.at[pl.ds(sid * RPW, RPW)], dsum_hbm.at[pl.ds(wid * RPW, RPW)])

    return sc_kernel(q_table, d_table, qs_flat, ds_flat, qpat, dpat, zeros)


def _tc_finish(q_sum, d_sum):
    def body(qs_ref, ds_ref, o_ref):
        q = jnp.tanh(qs_ref[...] * (1.0 / QL))
        d = jnp.tanh(ds_ref[...] * (1.0 / DL))
        qn = jnp.maximum(jnp.sqrt(jnp.sum(q * q, axis=1, keepdims=True)), 1e-12)
        dn = jnp.maximum(jnp.sqrt(jnp.sum(d * d, axis=1, keepdims=True)), 1e-12)
        o_ref[...] = jnp.sum((q / qn) * (d / dn), axis=1)

    return pl.pallas_call(
        body,
        out_shape=jax.ShapeDtypeStruct((B,), jnp.float32),
    )(q_sum, d_sum)


def kernel(qs, ds, q_table, d_table):
    qs_flat = qs.reshape(-1)
    ds_flat = ds.reshape(-1)
    qpat = jnp.asarray(_QPAT)
    dpat = jnp.asarray(_DPAT)
    zeros = jnp.zeros((RPW, D), jnp.float32)
    q_sum, d_sum = _sc_pool(q_table, d_table, qs_flat, ds_flat, qpat, dpat, zeros)
    return _tc_finish(q_sum, d_sum)


# speedup vs baseline: 1.3632x; 1.1057x over previous
import functools

import numpy as np
import jax
import jax.numpy as jnp
from jax import lax
from jax.experimental import pallas as pl
from jax.experimental.pallas import tpu as pltpu
from jax.experimental.pallas import tpu_sc as plsc

B = 4096
D = 64
QL = 20
DL = 200
NC = 2
NS = 16
NW = NC * NS
RPW = B // NW
QIW = RPW * QL
DIW = RPW * DL
CH = 128
QCH = QIW // CH
DCH = DIW // CH
NB = 4


def _seg_pattern(n_chunks: int, seg_len: int) -> np.ndarray:
    pos = np.arange(n_chunks * CH, dtype=np.int32)
    base = (pos // seg_len).reshape(1, n_chunks, CH)
    offs = (np.arange(NS, dtype=np.int32) * RPW).reshape(NS, 1, 1)
    return base + offs


_QPAT = _seg_pattern(QCH, QL)
_DPAT = _seg_pattern(DCH, DL)


def _sc_pool(table, idx_flat, pat, zeros, n_chunks):
    ipw = n_chunks * CH
    mesh = plsc.VectorSubcoreMesh(core_axis_name="c", subcore_axis_name="s")

    @functools.partial(
        pl.kernel,
        out_type=jax.ShapeDtypeStruct((B, D), jnp.float32),
        mesh=mesh,
        compiler_params=pltpu.CompilerParams(use_tc_tiling_on_sc=False),
        scratch_types=[
            pltpu.VMEM((ipw,), jnp.int32),
            pltpu.VMEM((n_chunks, CH), jnp.int32),
            pltpu.VMEM((NB, CH, D), jnp.float32),
            pltpu.VMEM_SHARED((NS * RPW, D), jnp.float32),
        ] + [pltpu.SemaphoreType.DMA] * (2 * NB),
    )
    def sc_kernel(tab_hbm, idx_hbm, pat_hbm, z_hbm, sum_hbm,
                  idx_v, pat_v, buf_v, acc, *sems):
        gsems, ssems = sems[:NB], sems[NB:]
        sid = lax.axis_index("s")
        wid = sid * NC + lax.axis_index("c")

        pltpu.sync_copy(idx_hbm.at[pl.ds(wid * ipw, ipw)], idx_v)
        pltpu.sync_copy(pat_hbm.at[sid], pat_v)
        pltpu.sync_copy(z_hbm, acc.at[pl.ds(sid * RPW, RPW)])

        def g_start(c, b):
            pltpu.async_copy(
                tab_hbm.at[idx_v.at[pl.ds(c * CH, CH)]], buf_v.at[b], gsems[b])

        def g_wait(b):
            pltpu.make_async_copy(
                tab_hbm.at[idx_v.at[pl.ds(0, CH)]], buf_v.at[b], gsems[b]
            ).wait()

        def s_start(c, b):
            pltpu.async_copy(
                buf_v.at[b], acc.at[pat_v.at[c]], ssems[b], add=True)

        def s_wait(b):
            pltpu.make_async_copy(
                buf_v.at[b], acc.at[pat_v.at[0]], ssems[b]).wait()

        for b in range(NB):
            g_start(b, b)

        @pl.loop(0, n_chunks // NB - 1)
        def _(g):
            for b in range(NB):
                g_wait(b)
                s_start(g * NB + b, b)
            for b in range(NB):
                s_wait(b)
                g_start((g + 1) * NB + b, b)

        last = (n_chunks // NB - 1) * NB
        for b in range(NB):
            g_wait(b)
            s_start(last + b, b)
        for b in range(NB):
            s_wait(b)

        pltpu.sync_copy(acc.at[pl.ds(sid * RPW, RPW)],
                        sum_hbm.at[pl.ds(wid * RPW, RPW)])

    return sc_kernel(table, idx_flat, pat, zeros)


def _tc_finish(q_sum, d_sum):
    def body(qs_ref, ds_ref, o_ref):
        q = jnp.tanh(qs_ref[...] * (1.0 / QL))
        d = jnp.tanh(ds_ref[...] * (1.0 / DL))
        qn = jnp.maximum(jnp.sqrt(jnp.sum(q * q, axis=1, keepdims=True)), 1e-12)
        dn = jnp.maximum(jnp.sqrt(jnp.sum(d * d, axis=1, keepdims=True)), 1e-12)
        o_ref[...] = jnp.sum((q / qn) * (d / dn), axis=1)

    return pl.pallas_call(
        body,
        out_shape=jax.ShapeDtypeStruct((B,), jnp.float32),
    )(q_sum, d_sum)


def kernel(qs, ds, q_table, d_table):
    zeros = jnp.zeros((RPW, D), jnp.float32)
    q_sum = _sc_pool(q_table, qs.reshape(-1), jnp.asarray(_QPAT), zeros, QCH)
    d_sum = _sc_pool(d_table, ds.reshape(-1), jnp.asarray(_DPAT), zeros, DCH)
    return _tc_finish(q_sum, d_sum)
